# baseline (speedup 1.0000x reference)
import functools

import jax
import jax.numpy as jnp
from jax import lax
from jax.experimental import pallas as pl
from jax.experimental.pallas import tpu as pltpu
from jax.experimental.pallas import tpu_sc as plsc

N = 16384
E = 32768
EH = E // 2
NG = 512
D = 64
DHE = 128

NC = 2
NS = 16
NW = NC * NS
EW = EH // NW
CHUNK = 128
NCHUNK = EW // CHUNK


@functools.cache
def _sc_kernels():
    mesh = plsc.VectorSubcoreMesh(
        core_axis_name="c", subcore_axis_name="s", num_cores=NC, num_subcores=NS
    )

    @functools.partial(
        pl.kernel,
        out_type=jax.ShapeDtypeStruct((EH, D), jnp.float32),
        mesh=mesh,
        scratch_types=[
            pltpu.VMEM((NCHUNK, CHUNK), jnp.int32),
            pltpu.VMEM((EW, D), jnp.float32),
            pltpu.SemaphoreType.DMA,
        ],
        compiler_params=pltpu.CompilerParams(use_tc_tiling_on_sc=False),
    )
    def sc_gather(table_hbm, idx_hbm, out_hbm, idx_v, rows_v, sem):
        cid = lax.axis_index("c")
        sid = lax.axis_index("s")
        wid = sid * NC + cid
        pltpu.sync_copy(idx_hbm.at[wid], idx_v)
        cps = []
        for j in range(NCHUNK):
            cps.append(
                pltpu.async_copy(
                    table_hbm.at[idx_v.at[j]],
                    rows_v.at[pl.ds(j * CHUNK, CHUNK)],
                    sem,
                )
            )
        for c in cps:
            c.wait()
        pltpu.sync_copy(rows_v, out_hbm.at[pl.ds(wid * EW, EW)])

    RPT = N // NS

    @functools.partial(
        pl.kernel,
        out_type=jax.ShapeDtypeStruct((NC, N, D), jnp.bfloat16),
        mesh=mesh,
        scratch_types=[
            pltpu.VMEM((NCHUNK, CHUNK), jnp.int32),
            pltpu.VMEM((EW, D), jnp.bfloat16),
            pltpu.VMEM_SHARED((N, D), jnp.bfloat16),
            pltpu.SemaphoreType.DMA,
            pltpu.SemaphoreType.DMA,
        ],
        compiler_params=pltpu.CompilerParams(use_tc_tiling_on_sc=False),
    )
    def sc_scatter_add(msg_hbm, idx_hbm, zeros_hbm, out_hbm,
                       idx_v, rows_v, acc_sh, sem, sem_s):
        cid = lax.axis_index("c")
        sid = lax.axis_index("s")
        wid = sid * NC + cid
        cp_m = pltpu.async_copy(msg_hbm.at[pl.ds(wid * EW, EW)], rows_v, sem)
        pltpu.sync_copy(idx_hbm.at[wid], idx_v)
        pltpu.sync_copy(zeros_hbm.at[pl.ds(0, RPT)],
                        acc_sh.at[pl.ds(sid * RPT, RPT)])
        plsc.subcore_barrier()
        cp_m.wait()
        scat = [
            pltpu.async_copy(
                rows_v.at[pl.ds(j * CHUNK, CHUNK)],
                acc_sh.at[idx_v.at[j]],
                sem_s,
                add=True,
            )
            for j in range(NCHUNK)
        ]
        for c in scat:
            c.wait()
        plsc.subcore_barrier()
        pltpu.sync_copy(
            acc_sh.at[pl.ds(sid * RPT, RPT)],
            out_hbm.at[cid, pl.ds(sid * RPT, RPT)],
        )

    return sc_gather, sc_scatter_add


TE = 512


def _edge_body(eaT_ref, xs_ref, w1T_ref, b1_ref, w2T_ref, B2T_ref, out_ref):
    eaT = eaT_ref[...]
    xsT = xs_ref[...].T
    h_eT = jnp.maximum(
        jnp.dot(w1T_ref[...], eaT, preferred_element_type=jnp.float32)
        + b1_ref[...],
        0.0,
    )
    WT = jnp.dot(
        w2T_ref[...],
        h_eT.astype(jnp.bfloat16),
        preferred_element_type=jnp.float32,
    )
    accT = jnp.dot(B2T_ref[...], xsT, preferred_element_type=jnp.float32)
    accT2 = jnp.zeros_like(accT)
    for i in range(0, D, 2):
        accT = accT + xsT[i : i + 1, :] * WT[i * D : (i + 1) * D, :]
        accT2 = accT2 + xsT[i + 1 : i + 2, :] * WT[(i + 1) * D : (i + 2) * D, :]
    out_ref[...] = (accT + accT2).astype(jnp.bfloat16).T


_edge_call = pl.pallas_call(
    _edge_body,
    grid=(EH // TE,),
    in_specs=[
        pl.BlockSpec((16, TE), lambda i: (0, i)),
        pl.BlockSpec((TE, D), lambda i: (i, 0)),
        pl.BlockSpec((DHE, 16), lambda i: (0, 0)),
        pl.BlockSpec((DHE, 1), lambda i: (0, 0)),
        pl.BlockSpec((D * D, DHE), lambda i: (0, 0)),
        pl.BlockSpec((D, D), lambda i: (0, 0)),
    ],
    out_specs=pl.BlockSpec((TE, D), lambda i: (i, 0)),
    out_shape=jax.ShapeDtypeStruct((EH, D), jnp.bfloat16),
    compiler_params=pltpu.CompilerParams(
        dimension_semantics=("parallel",),
    ),
)


TN = 2048


def _node_body(agga_ref, aggb_ref, h_ref, root_ref, bias_ref, batch_ref,
               h_out_ref, pool_ref):
    step = pl.program_id(0)
    h_new = jnp.maximum(
        agga_ref[0].astype(jnp.float32)
        + agga_ref[1].astype(jnp.float32)
        + aggb_ref[0].astype(jnp.float32)
        + aggb_ref[1].astype(jnp.float32)
        + jnp.dot(h_ref[...], root_ref[...], preferred_element_type=jnp.float32)
        + bias_ref[...],
        0.0,
    )
    h_out_ref[...] = h_new
    bid = batch_ref[0]
    gids = lax.broadcasted_iota(jnp.int32, (NG, TN), 0)
    mask = (bid == gids).astype(jnp.float32)
    part = jnp.dot(mask, h_new, preferred_element_type=jnp.float32)

    @pl.when(step == 0)
    def _():
        pool_ref[...] = jnp.zeros_like(pool_ref)

    pool_ref[...] += part


_node_call = pl.pallas_call(
    _node_body,
    grid=(N // TN,),
    in_specs=[
        pl.BlockSpec((2, TN, D), lambda i: (0, i, 0)),
        pl.BlockSpec((2, TN, D), lambda i: (0, i, 0)),
        pl.BlockSpec((TN, D), lambda i: (i, 0)),
        pl.BlockSpec((D, D), lambda i: (0, 0)),
        pl.BlockSpec((1, D), lambda i: (0, 0)),
        pl.BlockSpec((1, 1, TN), lambda i: (i, 0, 0)),
    ],
    out_specs=[
        pl.BlockSpec((TN, D), lambda i: (i, 0)),
        pl.BlockSpec((NG, D), lambda i: (0, 0)),
    ],
    out_shape=[
        jax.ShapeDtypeStruct((N, D), jnp.float32),
        jax.ShapeDtypeStruct((NG, D), jnp.float32),
    ],
    compiler_params=pltpu.CompilerParams(
        dimension_semantics=("arbitrary",),
    ),
)


def _final_body(pc_ref, lw_ref, lb_ref, out_ref):
    out_ref[...] = (
        jnp.dot(pc_ref[...], lw_ref[...], preferred_element_type=jnp.float32)
        + lb_ref[...]
    )


_final_call = pl.pallas_call(
    _final_body,
    out_shape=jax.ShapeDtypeStruct((NG, 256), jnp.float32),
)


def kernel(x, edge_index, edge_attr, batch,
           mlp_w1_0, mlp_b1_0, mlp_w2_0, mlp_b2_0, root_0, bias_0,
           mlp_w1_1, mlp_b1_1, mlp_w2_1, mlp_b2_1, root_1, bias_1,
           lin_w, lin_b):
    src = edge_index[0]
    dst = edge_index[1]
    srcs = [src[:EH].reshape(NW, NCHUNK, CHUNK),
            src[EH:].reshape(NW, NCHUNK, CHUNK)]
    dsts = [dst[:EH].reshape(NW, NCHUNK, CHUNK),
            dst[EH:].reshape(NW, NCHUNK, CHUNK)]
    eaTs = [edge_attr[:EH].T, edge_attr[EH:].T]
    zeros = jnp.zeros((N // NS, D), jnp.bfloat16)
    batch_r = batch.reshape(N // TN, 1, TN)

    layers = [
        (mlp_w1_0, mlp_b1_0, mlp_w2_0, mlp_b2_0, root_0, bias_0),
        (mlp_w1_1, mlp_b1_1, mlp_w2_1, mlp_b2_1, root_1, bias_1),
    ]
    sc_gather, sc_scatter_add = _sc_kernels()
    h = x
    pools = []
    for (w1, b1, w2, b2, root, bias) in layers:
        w1T = w1.T
        b1c = b1.reshape(DHE, 1)
        w2T = w2.T.astype(jnp.bfloat16)
        B2T = b2.reshape(D, D).T
        xsA = sc_gather(h, srcs[0])
        xsB = sc_gather(h, srcs[1])
        msgA = _edge_call(eaTs[0], xsA, w1T, b1c, w2T, B2T)
        aggA = sc_scatter_add(msgA, dsts[0], zeros)
        msgB = _edge_call(eaTs[1], xsB, w1T, b1c, w2T, B2T)
        aggB = sc_scatter_add(msgB, dsts[1], zeros)
        h, pool = _node_call(aggA, aggB, h, root, bias.reshape(1, D), batch_r)
        pools.append(pool)
    pc = jnp.concatenate(pools, axis=1)
    return _final_call(pc, lin_w, lin_b.reshape(1, 256))


# speedup vs baseline: 1.0997x; 1.0997x over previous
import functools

import jax
import jax.numpy as jnp
from jax import lax
from jax.experimental import pallas as pl
from jax.experimental.pallas import tpu as pltpu
from jax.experimental.pallas import tpu_sc as plsc

N = 16384
E = 32768
NG = 512
D = 64
DHE = 128

NC = 2
NS = 16
NW = NC * NS
EW = E // NW
CHUNK = 128
NCHUNK = EW // CHUNK

@functools.cache
def _sc_kernels():
    mesh = plsc.VectorSubcoreMesh(
        core_axis_name="c", subcore_axis_name="s", num_cores=NC, num_subcores=NS
    )

    @functools.partial(
        pl.kernel,
        out_type=jax.ShapeDtypeStruct((E, D), jnp.float32),
        mesh=mesh,
        scratch_types=[
            pltpu.VMEM((NCHUNK, CHUNK), jnp.int32),
            pltpu.VMEM((EW, D), jnp.float32),
            pltpu.SemaphoreType.DMA,
        ],
        compiler_params=pltpu.CompilerParams(use_tc_tiling_on_sc=False),
    )
    def sc_gather(table_hbm, idx_hbm, out_hbm, idx_v, rows_v, sem):
        cid = lax.axis_index("c")
        sid = lax.axis_index("s")
        wid = sid * NC + cid
        pltpu.sync_copy(idx_hbm.at[wid], idx_v)
        cps = []
        for j in range(NCHUNK):
            cps.append(
                pltpu.async_copy(
                    table_hbm.at[idx_v.at[j]],
                    rows_v.at[pl.ds(j * CHUNK, CHUNK)],
                    sem,
                )
            )
        for c in cps:
            c.wait()
        pltpu.sync_copy(rows_v, out_hbm.at[pl.ds(wid * EW, EW)])

    RPT = N // NS

    @functools.partial(
        pl.kernel,
        out_type=jax.ShapeDtypeStruct((NC, N, D), jnp.bfloat16),
        mesh=mesh,
        scratch_types=[
            pltpu.VMEM((NCHUNK, CHUNK), jnp.int32),
            pltpu.VMEM((EW, D), jnp.bfloat16),
            pltpu.VMEM_SHARED((N, D), jnp.bfloat16),
            pltpu.SemaphoreType.DMA,
            pltpu.SemaphoreType.DMA,
        ],
        compiler_params=pltpu.CompilerParams(use_tc_tiling_on_sc=False),
    )
    def sc_scatter_add(msg_hbm, idx_hbm, zeros_hbm, out_hbm,
                       idx_v, rows_v, acc_sh, sem, sem_s):
        cid = lax.axis_index("c")
        sid = lax.axis_index("s")
        wid = sid * NC + cid
        cp_m = pltpu.async_copy(msg_hbm.at[pl.ds(wid * EW, EW)], rows_v, sem)
        pltpu.sync_copy(idx_hbm.at[wid], idx_v)
        pltpu.sync_copy(zeros_hbm.at[pl.ds(0, RPT)],
                        acc_sh.at[pl.ds(sid * RPT, RPT)])
        plsc.subcore_barrier()
        cp_m.wait()
        scat = [
            pltpu.async_copy(
                rows_v.at[pl.ds(j * CHUNK, CHUNK)],
                acc_sh.at[idx_v.at[j]],
                sem_s,
                add=True,
            )
            for j in range(NCHUNK)
        ]
        for c in scat:
            c.wait()
        plsc.subcore_barrier()
        pltpu.sync_copy(
            acc_sh.at[pl.ds(sid * RPT, RPT)],
            out_hbm.at[cid, pl.ds(sid * RPT, RPT)],
        )

    return sc_gather, sc_scatter_add


TE = 1024


def _edge_body(eaT_ref, xs_ref, w1T_ref, b1_ref, w2T_ref, B2T_ref, out_ref):
    eaT = eaT_ref[...]
    xsT = xs_ref[...].T
    h_eT = jnp.maximum(
        jnp.dot(w1T_ref[...], eaT, preferred_element_type=jnp.float32)
        + b1_ref[...],
        0.0,
    )
    WT = jnp.dot(
        w2T_ref[...],
        h_eT.astype(jnp.bfloat16),
        preferred_element_type=jnp.float32,
    )
    accT = jnp.dot(B2T_ref[...], xsT, preferred_element_type=jnp.float32)
    accT2 = jnp.zeros_like(accT)
    for i in range(0, D, 2):
        accT = accT + xsT[i : i + 1, :] * WT[i * D : (i + 1) * D, :]
        accT2 = accT2 + xsT[i + 1 : i + 2, :] * WT[(i + 1) * D : (i + 2) * D, :]
    out_ref[...] = (accT + accT2).astype(jnp.bfloat16).T


_edge_call = pl.pallas_call(
    _edge_body,
    grid=(E // TE,),
    in_specs=[
        pl.BlockSpec((16, TE), lambda i: (0, i)),
        pl.BlockSpec((TE, D), lambda i: (i, 0)),
        pl.BlockSpec((DHE, 16), lambda i: (0, 0)),
        pl.BlockSpec((DHE, 1), lambda i: (0, 0)),
        pl.BlockSpec((D * D, DHE), lambda i: (0, 0)),
        pl.BlockSpec((D, D), lambda i: (0, 0)),
    ],
    out_specs=pl.BlockSpec((TE, D), lambda i: (i, 0)),
    out_shape=jax.ShapeDtypeStruct((E, D), jnp.bfloat16),
    compiler_params=pltpu.CompilerParams(
        dimension_semantics=("parallel",),
    ),
)


TN = 2048


def _node_body(agg_ref, h_ref, root_ref, bias_ref, batch_ref,
               h_out_ref, pool_ref):
    step = pl.program_id(0)
    h_new = jnp.maximum(
        agg_ref[0].astype(jnp.float32)
        + agg_ref[1].astype(jnp.float32)
        + jnp.dot(h_ref[...], root_ref[...], preferred_element_type=jnp.float32)
        + bias_ref[...],
        0.0,
    )
    h_out_ref[...] = h_new
    bid = batch_ref[0]
    gids = lax.broadcasted_iota(jnp.int32, (NG, TN), 0)
    mask = (bid == gids).astype(jnp.float32)
    part = jnp.dot(mask, h_new, preferred_element_type=jnp.float32)

    @pl.when(step == 0)
    def _():
        pool_ref[...] = jnp.zeros_like(pool_ref)

    pool_ref[...] += part


_node_call = pl.pallas_call(
    _node_body,
    grid=(N // TN,),
    in_specs=[
        pl.BlockSpec((2, TN, D), lambda i: (0, i, 0)),
        pl.BlockSpec((TN, D), lambda i: (i, 0)),
        pl.BlockSpec((D, D), lambda i: (0, 0)),
        pl.BlockSpec((1, D), lambda i: (0, 0)),
        pl.BlockSpec((1, 1, TN), lambda i: (i, 0, 0)),
    ],
    out_specs=[
        pl.BlockSpec((TN, D), lambda i: (i, 0)),
        pl.BlockSpec((NG, D), lambda i: (0, 0)),
    ],
    out_shape=[
        jax.ShapeDtypeStruct((N, D), jnp.float32),
        jax.ShapeDtypeStruct((NG, D), jnp.float32),
    ],
    compiler_params=pltpu.CompilerParams(
        dimension_semantics=("arbitrary",),
    ),
)


def _final_body(pc_ref, lw_ref, lb_ref, out_ref):
    out_ref[...] = (
        jnp.dot(pc_ref[...], lw_ref[...], preferred_element_type=jnp.float32)
        + lb_ref[...]
    )


_final_call = pl.pallas_call(
    _final_body,
    out_shape=jax.ShapeDtypeStruct((NG, 256), jnp.float32),
)


def kernel(x, edge_index, edge_attr, batch,
           mlp_w1_0, mlp_b1_0, mlp_w2_0, mlp_b2_0, root_0, bias_0,
           mlp_w1_1, mlp_b1_1, mlp_w2_1, mlp_b2_1, root_1, bias_1,
           lin_w, lin_b):
    src = edge_index[0].reshape(NW, NCHUNK, CHUNK)
    dst = edge_index[1].reshape(NW, NCHUNK, CHUNK)
    zeros = jnp.zeros((N // NS, D), jnp.bfloat16)
    batch_r = batch.reshape(N // TN, 1, TN)

    layers = [
        (mlp_w1_0, mlp_b1_0, mlp_w2_0, mlp_b2_0, root_0, bias_0),
        (mlp_w1_1, mlp_b1_1, mlp_w2_1, mlp_b2_1, root_1, bias_1),
    ]
    sc_gather, sc_scatter_add = _sc_kernels()
    h = x
    pools = []
    for (w1, b1, w2, b2, root, bias) in layers:
        xs = sc_gather(h, src)
        msg = _edge_call(edge_attr.T, xs, w1.T, b1.reshape(DHE, 1),
                         w2.T.astype(jnp.bfloat16), b2.reshape(D, D).T)
        agg = sc_scatter_add(msg, dst, zeros)
        h, pool = _node_call(agg, h, root, bias.reshape(1, D), batch_r)
        pools.append(pool)
    pc = jnp.concatenate(pools, axis=1)
    return _final_call(pc, lin_w, lin_b.reshape(1, 256))


# speedup vs baseline: 1.1370x; 1.0339x over previous
import functools

import jax
import jax.numpy as jnp
from jax import lax
from jax.experimental import pallas as pl
from jax.experimental.pallas import tpu as pltpu
from jax.experimental.pallas import tpu_sc as plsc

N = 16384
E = 32768
NG = 512
D = 64
DHE = 128

NC = 2
NS = 16
NW = NC * NS
EW = E // NW
CHUNK = 128
NCHUNK = EW // CHUNK

@functools.cache
def _sc_kernels():
    mesh = plsc.VectorSubcoreMesh(
        core_axis_name="c", subcore_axis_name="s", num_cores=NC, num_subcores=NS
    )

    @functools.partial(
        pl.kernel,
        out_type=jax.ShapeDtypeStruct((E, D), jnp.float32),
        mesh=mesh,
        scratch_types=[
            pltpu.VMEM((NCHUNK, CHUNK), jnp.int32),
            pltpu.VMEM((EW, D), jnp.float32),
            pltpu.SemaphoreType.DMA,
        ],
        compiler_params=pltpu.CompilerParams(use_tc_tiling_on_sc=False),
    )
    def sc_gather(table_hbm, idx_hbm, out_hbm, idx_v, rows_v, sem):
        cid = lax.axis_index("c")
        sid = lax.axis_index("s")
        wid = sid * NC + cid
        pltpu.sync_copy(idx_hbm.at[wid], idx_v)
        cps = []
        for j in range(NCHUNK):
            cps.append(
                pltpu.async_copy(
                    table_hbm.at[idx_v.at[j]],
                    rows_v.at[pl.ds(j * CHUNK, CHUNK)],
                    sem,
                )
            )
        for c in cps:
            c.wait()
        pltpu.sync_copy(rows_v, out_hbm.at[pl.ds(wid * EW, EW)])

    RPT = N // NS

    @functools.partial(
        pl.kernel,
        out_type=jax.ShapeDtypeStruct((NC, N, D), jnp.bfloat16),
        mesh=mesh,
        scratch_types=[
            pltpu.VMEM((NCHUNK, CHUNK), jnp.int32),
            pltpu.VMEM((EW, D), jnp.bfloat16),
            pltpu.VMEM_SHARED((N, D), jnp.bfloat16),
            pltpu.SemaphoreType.DMA,
            pltpu.SemaphoreType.DMA,
        ],
        compiler_params=pltpu.CompilerParams(use_tc_tiling_on_sc=False),
    )
    def sc_scatter_add(msg_hbm, idx_hbm, zeros_hbm, out_hbm,
                       idx_v, rows_v, acc_sh, sem, sem_s):
        cid = lax.axis_index("c")
        sid = lax.axis_index("s")
        wid = sid * NC + cid
        cp_m = pltpu.async_copy(msg_hbm.at[pl.ds(wid * EW, EW)], rows_v, sem)
        pltpu.sync_copy(idx_hbm.at[wid], idx_v)
        pltpu.sync_copy(zeros_hbm.at[pl.ds(0, RPT)],
                        acc_sh.at[pl.ds(sid * RPT, RPT)])
        plsc.subcore_barrier()
        cp_m.wait()
        scat = [
            pltpu.async_copy(
                rows_v.at[pl.ds(j * CHUNK, CHUNK)],
                acc_sh.at[idx_v.at[j]],
                sem_s,
                add=True,
            )
            for j in range(NCHUNK)
        ]
        for c in scat:
            c.wait()
        plsc.subcore_barrier()
        pltpu.sync_copy(
            acc_sh.at[pl.ds(sid * RPT, RPT)],
            out_hbm.at[cid, pl.ds(sid * RPT, RPT)],
        )

    return sc_gather, sc_scatter_add


TE = 2048


def _edge_body(eaT_ref, xs_ref, w1T_ref, b1_ref, w2T_ref, B2T_ref, out_ref):
    eaT = eaT_ref[...]
    xsT = xs_ref[...].T
    h_eT = jnp.maximum(
        jnp.dot(w1T_ref[...], eaT, preferred_element_type=jnp.float32)
        + b1_ref[...],
        0.0,
    )
    WT = jnp.dot(
        w2T_ref[...],
        h_eT.astype(jnp.bfloat16),
        preferred_element_type=jnp.float32,
    )
    accT = jnp.dot(B2T_ref[...], xsT, preferred_element_type=jnp.float32)
    accT2 = jnp.zeros_like(accT)
    for i in range(0, D, 2):
        accT = accT + xsT[i : i + 1, :] * WT[i * D : (i + 1) * D, :]
        accT2 = accT2 + xsT[i + 1 : i + 2, :] * WT[(i + 1) * D : (i + 2) * D, :]
    out_ref[...] = (accT + accT2).astype(jnp.bfloat16).T


_edge_call = pl.pallas_call(
    _edge_body,
    grid=(E // TE,),
    in_specs=[
        pl.BlockSpec((16, TE), lambda i: (0, i)),
        pl.BlockSpec((TE, D), lambda i: (i, 0)),
        pl.BlockSpec((DHE, 16), lambda i: (0, 0)),
        pl.BlockSpec((DHE, 1), lambda i: (0, 0)),
        pl.BlockSpec((D * D, DHE), lambda i: (0, 0)),
        pl.BlockSpec((D, D), lambda i: (0, 0)),
    ],
    out_specs=pl.BlockSpec((TE, D), lambda i: (i, 0)),
    out_shape=jax.ShapeDtypeStruct((E, D), jnp.bfloat16),
    compiler_params=pltpu.CompilerParams(
        dimension_semantics=("parallel",),
    ),
)


TN = 2048


def _node_body(agg_ref, h_ref, root_ref, bias_ref, batch_ref,
               h_out_ref, pool_ref):
    step = pl.program_id(0)
    h_new = jnp.maximum(
        agg_ref[0].astype(jnp.float32)
        + agg_ref[1].astype(jnp.float32)
        + jnp.dot(h_ref[...], root_ref[...], preferred_element_type=jnp.float32)
        + bias_ref[...],
        0.0,
    )
    h_out_ref[...] = h_new
    bid = batch_ref[0]
    gids = lax.broadcasted_iota(jnp.int32, (NG, TN), 0)
    mask = (bid == gids).astype(jnp.float32)
    part = jnp.dot(mask, h_new, preferred_element_type=jnp.float32)

    @pl.when(step == 0)
    def _():
        pool_ref[...] = jnp.zeros_like(pool_ref)

    pool_ref[...] += part


_node_call = pl.pallas_call(
    _node_body,
    grid=(N // TN,),
    in_specs=[
        pl.BlockSpec((2, TN, D), lambda i: (0, i, 0)),
        pl.BlockSpec((TN, D), lambda i: (i, 0)),
        pl.BlockSpec((D, D), lambda i: (0, 0)),
        pl.BlockSpec((1, D), lambda i: (0, 0)),
        pl.BlockSpec((1, 1, TN), lambda i: (i, 0, 0)),
    ],
    out_specs=[
        pl.BlockSpec((TN, D), lambda i: (i, 0)),
        pl.BlockSpec((NG, D), lambda i: (0, 0)),
    ],
    out_shape=[
        jax.ShapeDtypeStruct((N, D), jnp.float32),
        jax.ShapeDtypeStruct((NG, D), jnp.float32),
    ],
    compiler_params=pltpu.CompilerParams(
        dimension_semantics=("arbitrary",),
    ),
)


def _final_body(pc_ref, lw_ref, lb_ref, out_ref):
    out_ref[...] = (
        jnp.dot(pc_ref[...], lw_ref[...], preferred_element_type=jnp.float32)
        + lb_ref[...]
    )


_final_call = pl.pallas_call(
    _final_body,
    out_shape=jax.ShapeDtypeStruct((NG, 256), jnp.float32),
)


def kernel(x, edge_index, edge_attr, batch,
           mlp_w1_0, mlp_b1_0, mlp_w2_0, mlp_b2_0, root_0, bias_0,
           mlp_w1_1, mlp_b1_1, mlp_w2_1, mlp_b2_1, root_1, bias_1,
           lin_w, lin_b):
    src = edge_index[0].reshape(NW, NCHUNK, CHUNK)
    dst = edge_index[1].reshape(NW, NCHUNK, CHUNK)
    zeros = jnp.zeros((N // NS, D), jnp.bfloat16)
    batch_r = batch.reshape(N // TN, 1, TN)

    layers = [
        (mlp_w1_0, mlp_b1_0, mlp_w2_0, mlp_b2_0, root_0, bias_0),
        (mlp_w1_1, mlp_b1_1, mlp_w2_1, mlp_b2_1, root_1, bias_1),
    ]
    sc_gather, sc_scatter_add = _sc_kernels()
    h = x
    pools = []
    for (w1, b1, w2, b2, root, bias) in layers:
        xs = sc_gather(h, src)
        msg = _edge_call(edge_attr.T, xs, w1.T, b1.reshape(DHE, 1),
                         w2.T.astype(jnp.bfloat16), b2.reshape(D, D).T)
        agg = sc_scatter_add(msg, dst, zeros)
        h, pool = _node_call(agg, h, root, bias.reshape(1, D), batch_r)
        pools.append(pool)
    pc = jnp.concatenate(pools, axis=1)
    return _final_call(pc, lin_w, lin_b.reshape(1, 256))
